# baseline (speedup 1.0000x reference)
import jax
import jax.numpy as jnp
import numpy as np
from jax import lax
from jax.experimental import pallas as pl
from jax.experimental.pallas import tpu as pltpu
from jax.experimental.pallas import tpu_sc as plsc

NBE = 320
NHIST = 2 * NBE
NPHASE = 4
SCALE = float(np.float32(NBE / 16.0))
LANES = 16
B = 32
P = 512 * 512
WIN = 8192
WROW = WIN // 512
NWIN = P // WIN
NCHUNK = NBE // LANES
RCHUNK = NHIST // LANES
ZCHUNK = LANES * NPHASE * NHIST // LANES


def _sc_body(seg_hbm, tgt_hbm, out_hbm, cnt_ref, sum_ref,
             seg0, seg1, tgt0, tgt1, red_cnt, red_sum, stage, sem0, sem1):
    img = lax.axis_index("s") * 2 + lax.axis_index("c")
    lane_base = lax.iota(jnp.int32, LANES) * (NPHASE * NHIST)
    ones = jnp.ones((LANES,), jnp.float32)
    zeros = jnp.zeros((LANES,), jnp.float32)
    segb = (seg0, seg1)
    tgtb = (tgt0, tgt1)
    sems = (sem0, sem1)

    def _start(w, b):
        c1 = pltpu.async_copy(seg_hbm.at[img, 0, pl.ds(w * WROW, WROW), :],
                              segb[b], sems[b])
        c2 = pltpu.async_copy(tgt_hbm.at[img, 0, pl.ds(w * WROW, WROW), :],
                              tgtb[b], sems[b])
        return c1, c2

    @plsc.parallel_loop(0, ZCHUNK, unroll=4)
    def _zero(i):
        cnt_ref[pl.ds(i * LANES, LANES)] = zeros
        sum_ref[pl.ds(i * LANES, LANES)] = zeros

    cur = _start(0, 0)
    nxt = _start(1, 1)
    macc = zeros
    for w in range(NWIN):
        b = w % 2
        cur[0].wait()
        cur[1].wait()
        seg_buf = segb[b]
        tgt_buf = tgtb[b]

        def _elem(i, macc):
            row = i >> 5
            col = (i & 31) * LANES
            lg = seg_buf[row, pl.ds(col, LANES)]
            tg = tgt_buf[row, pl.ds(col, LANES)]
            pos = tg > 0
            e = jnp.maximum(jnp.where(pos, 1.0 - lg, 1.0 + lg), 0.0)
            bk = jnp.minimum((e * SCALE).astype(jnp.int32), NBE - 1)
            idx = lane_base + (i % NPHASE) * NHIST + bk + tg * NBE
            plsc.addupdate_scatter(cnt_ref, [idx], ones)
            plsc.addupdate_scatter(sum_ref, [idx], e)
            return jnp.maximum(macc, e)

        macc = plsc.parallel_loop(0, WIN // LANES, unroll=NPHASE,
                                  carry=macc)(_elem)
        cur = nxt
        if w + 2 < NWIN:
            nxt = _start(w + 2, b)

    def _reduce(i, gv):
        base = i * LANES
        cn = zeros
        sn = zeros
        for r in range(LANES * NPHASE):
            cn = cn + cnt_ref[pl.ds(r * NHIST + base, LANES)]
            sn = sn + sum_ref[pl.ds(r * NHIST + base, LANES)]
        red_cnt[pl.ds(base, LANES)] = cn
        red_sum[pl.ds(base, LANES)] = sn
        return gv + jnp.where(i >= NCHUNK, cn, zeros)

    gvec = lax.fori_loop(0, RCHUNK, _reduce, zeros)
    G = jnp.sum(gvec)

    def _chunk(i, carry):
        cm, cpc, lossacc = carry
        v0 = (NCHUNK - 1 - i) * LANES
        cn = red_cnt[pl.ds(v0, LANES)]
        cp = red_cnt[pl.ds(NBE + v0, LANES)]
        sn = red_sum[pl.ds(v0, LANES)]
        sp = red_sum[pl.ds(NBE + v0, LANES)]
        m = cn + cp
        pre_m = plsc.cumsum(m)
        pre_p = plsc.cumsum(cp)
        tot_m = jnp.sum(m)
        tot_p = jnp.sum(cp)
        nex = cm + (tot_m - pre_m)
        pex = cpc + (tot_p - pre_p)
        nb = nex - pex
        pos = sp / (G + nb + 0.5 * cn)
        neg = (G - pex - 0.5 * cp) * sn / ((G + nb) * (G + nb + cn))
        return cm + tot_m, cpc + tot_p, lossacc + pos + neg

    _, _, lossacc = lax.fori_loop(0, NCHUNK, _chunk, (0.0, 0.0, zeros))

    stage[pl.ds(0, LANES)] = lossacc
    stage[pl.ds(LANES, LANES)] = gvec
    stage[pl.ds(2 * LANES, LANES)] = macc
    pltpu.sync_copy(stage, out_hbm.at[img])


def _combine_body(sc_ref, pp_ref, o_ref):
    sc = sc_ref[...]
    lossv = jnp.sum(sc[:, 0:16], axis=1, keepdims=True)
    gv = jnp.sum(sc[:, 16:32], axis=1, keepdims=True)
    maxv = jnp.max(sc[:, 32:48], axis=1, keepdims=True)
    lov = jnp.where(gv > 0.0, lossv, maxv)
    lovasz = jnp.sum(lov) * (0.5 / B)
    p = jnp.clip(pp_ref[...], 1e-12, 1.0 - 1e-12)
    trg = (gv > 0.0).astype(jnp.float32)
    bce = -(trg * jnp.log(p) + (1.0 - trg) * jnp.log(1.0 - p))
    o_ref[...] = (jnp.sum(bce) * (0.5 / B) + lovasz).reshape(1, 1)


@jax.jit
def kernel(segm, prob_pred, target):
    seg2 = segm
    tgt2 = target.astype(jnp.int32)
    sc_fn = pl.kernel(
        _sc_body,
        out_type=jax.ShapeDtypeStruct((B, 3 * LANES), jnp.float32),
        mesh=plsc.VectorSubcoreMesh(core_axis_name="c", subcore_axis_name="s"),
        scratch_types=[
            pltpu.VMEM((LANES * NPHASE * NHIST,), jnp.float32),
            pltpu.VMEM((LANES * NPHASE * NHIST,), jnp.float32),
            pltpu.VMEM((WROW, 512), jnp.float32),
            pltpu.VMEM((WROW, 512), jnp.float32),
            pltpu.VMEM((WROW, 512), jnp.int32),
            pltpu.VMEM((WROW, 512), jnp.int32),
            pltpu.VMEM((NHIST,), jnp.float32),
            pltpu.VMEM((NHIST,), jnp.float32),
            pltpu.VMEM((3 * LANES,), jnp.float32),
            pltpu.SemaphoreType.DMA,
            pltpu.SemaphoreType.DMA,
        ],
        compiler_params=pltpu.CompilerParams(needs_layout_passes=False,
                                             use_tc_tiling_on_sc=True),
    )
    sc_out = sc_fn(seg2, tgt2)
    out = pl.pallas_call(
        _combine_body,
        out_shape=jax.ShapeDtypeStruct((1, 1), jnp.float32),
    )(sc_out, prob_pred.reshape(B, 1))
    return out[0, 0]


# speedup vs baseline: 1.0040x; 1.0040x over previous
import jax
import jax.numpy as jnp
import numpy as np
from jax import lax
from jax.experimental import pallas as pl
from jax.experimental.pallas import tpu as pltpu
from jax.experimental.pallas import tpu_sc as plsc

NBE = 320
NHIST = 2 * NBE
NPHASE = 4
SCALE = float(np.float32(NBE / 16.0))
CLAMP = float((NBE - 1) / np.float32(NBE / 16.0))
LANES = 16
B = 32
P = 512 * 512
WIN = 8192
WROW = WIN // 512
NWIN = P // WIN
NCHUNK = NBE // LANES
RCHUNK = NHIST // LANES
ZCHUNK = LANES * NPHASE * NHIST // LANES


def _sc_body(seg_hbm, tgt_hbm, out_hbm, cnt_ref, sum_ref,
             seg0, seg1, tgt0, tgt1, red_cnt, red_sum, stage, sem0, sem1):
    img = lax.axis_index("s") * 2 + lax.axis_index("c")
    lane_base = lax.iota(jnp.int32, LANES) * (NPHASE * NHIST)
    ones = jnp.ones((LANES,), jnp.float32)
    zeros = jnp.zeros((LANES,), jnp.float32)
    segb = (seg0, seg1)
    tgtb = (tgt0, tgt1)
    sems = (sem0, sem1)

    def _start(w, b):
        c1 = pltpu.async_copy(seg_hbm.at[img, 0, pl.ds(w * WROW, WROW), :],
                              segb[b], sems[b])
        c2 = pltpu.async_copy(tgt_hbm.at[img, 0, pl.ds(w * WROW, WROW), :],
                              tgtb[b], sems[b])
        return c1, c2

    @plsc.parallel_loop(0, ZCHUNK, unroll=4)
    def _zero(i):
        cnt_ref[pl.ds(i * LANES, LANES)] = zeros
        sum_ref[pl.ds(i * LANES, LANES)] = zeros

    cur = _start(0, 0)
    nxt = _start(1, 1)
    macc = zeros
    for w in range(NWIN):
        b = w % 2
        cur[0].wait()
        cur[1].wait()
        seg_buf = segb[b]
        tgt_buf = tgtb[b]

        def _elem(i, macc):
            row = i >> 5
            col = (i & 31) * LANES
            lg = seg_buf[row, pl.ds(col, LANES)]
            tg = tgt_buf[row, pl.ds(col, LANES)]
            pos = tg > 0
            e = jnp.maximum(jnp.where(pos, 1.0 - lg, 1.0 + lg), 0.0)
            bk = (jnp.minimum(e, CLAMP) * SCALE).astype(jnp.int32)
            idx = lane_base + (i % NPHASE) * NHIST + bk + tg * NBE
            plsc.addupdate_scatter(cnt_ref, [idx], ones)
            plsc.addupdate_scatter(sum_ref, [idx], e)
            return jnp.maximum(macc, e)

        macc = plsc.parallel_loop(0, WIN // LANES, unroll=NPHASE,
                                  carry=macc)(_elem)
        cur = nxt
        if w + 2 < NWIN:
            nxt = _start(w + 2, b)

    def _reduce(i, gv):
        base = i * LANES
        cn = zeros
        sn = zeros
        for r in range(LANES * NPHASE):
            cn = cn + cnt_ref[pl.ds(r * NHIST + base, LANES)]
            sn = sn + sum_ref[pl.ds(r * NHIST + base, LANES)]
        red_cnt[pl.ds(base, LANES)] = cn
        red_sum[pl.ds(base, LANES)] = sn
        return gv + jnp.where(i >= NCHUNK, cn, zeros)

    gvec = lax.fori_loop(0, RCHUNK, _reduce, zeros)
    G = jnp.sum(gvec)

    def _chunk(i, carry):
        cm, cpc, lossacc = carry
        v0 = (NCHUNK - 1 - i) * LANES
        cn = red_cnt[pl.ds(v0, LANES)]
        cp = red_cnt[pl.ds(NBE + v0, LANES)]
        sn = red_sum[pl.ds(v0, LANES)]
        sp = red_sum[pl.ds(NBE + v0, LANES)]
        m = cn + cp
        pre_m = plsc.cumsum(m)
        pre_p = plsc.cumsum(cp)
        tot_m = jnp.sum(m)
        tot_p = jnp.sum(cp)
        nex = cm + (tot_m - pre_m)
        pex = cpc + (tot_p - pre_p)
        nb = nex - pex
        pos = sp / (G + nb + 0.5 * cn)
        neg = (G - pex - 0.5 * cp) * sn / ((G + nb) * (G + nb + cn))
        return cm + tot_m, cpc + tot_p, lossacc + pos + neg

    _, _, lossacc = lax.fori_loop(0, NCHUNK, _chunk, (0.0, 0.0, zeros))

    stage[pl.ds(0, LANES)] = lossacc
    stage[pl.ds(LANES, LANES)] = gvec
    stage[pl.ds(2 * LANES, LANES)] = macc
    pltpu.sync_copy(stage, out_hbm.at[img])


def _combine_body(sc_ref, pp_ref, o_ref):
    sc = sc_ref[...]
    lossv = jnp.sum(sc[:, 0:16], axis=1, keepdims=True)
    gv = jnp.sum(sc[:, 16:32], axis=1, keepdims=True)
    maxv = jnp.max(sc[:, 32:48], axis=1, keepdims=True)
    lov = jnp.where(gv > 0.0, lossv, maxv)
    lovasz = jnp.sum(lov) * (0.5 / B)
    p = jnp.clip(pp_ref[...], 1e-12, 1.0 - 1e-12)
    trg = (gv > 0.0).astype(jnp.float32)
    bce = -(trg * jnp.log(p) + (1.0 - trg) * jnp.log(1.0 - p))
    o_ref[...] = (jnp.sum(bce) * (0.5 / B) + lovasz).reshape(1, 1)


@jax.jit
def kernel(segm, prob_pred, target):
    seg2 = segm
    tgt2 = target.astype(jnp.int32)
    sc_fn = pl.kernel(
        _sc_body,
        out_type=jax.ShapeDtypeStruct((B, 3 * LANES), jnp.float32),
        mesh=plsc.VectorSubcoreMesh(core_axis_name="c", subcore_axis_name="s"),
        scratch_types=[
            pltpu.VMEM((LANES * NPHASE * NHIST,), jnp.float32),
            pltpu.VMEM((LANES * NPHASE * NHIST,), jnp.float32),
            pltpu.VMEM((WROW, 512), jnp.float32),
            pltpu.VMEM((WROW, 512), jnp.float32),
            pltpu.VMEM((WROW, 512), jnp.int32),
            pltpu.VMEM((WROW, 512), jnp.int32),
            pltpu.VMEM((NHIST,), jnp.float32),
            pltpu.VMEM((NHIST,), jnp.float32),
            pltpu.VMEM((3 * LANES,), jnp.float32),
            pltpu.SemaphoreType.DMA,
            pltpu.SemaphoreType.DMA,
        ],
        compiler_params=pltpu.CompilerParams(needs_layout_passes=False,
                                             use_tc_tiling_on_sc=True),
    )
    sc_out = sc_fn(seg2, tgt2)
    out = pl.pallas_call(
        _combine_body,
        out_shape=jax.ShapeDtypeStruct((1, 1), jnp.float32),
    )(sc_out, prob_pred.reshape(B, 1))
    return out[0, 0]


# speedup vs baseline: 1.0447x; 1.0405x over previous
import jax
import jax.numpy as jnp
import numpy as np
from jax import lax
from jax.experimental import pallas as pl
from jax.experimental.pallas import tpu as pltpu
from jax.experimental.pallas import tpu_sc as plsc

NBE = 224
NHIST = 2 * NBE
NPHASE = 4
SCALE = float(np.float32(NBE / 16.0))
CLAMP = float((NBE - 1) / np.float32(NBE / 16.0))
LANES = 16
B = 32
P = 512 * 512
WIN = 16384
WROW = WIN // 512
NWIN = P // WIN
NCHUNK = NBE // LANES
RCHUNK = NHIST // LANES
ZCHUNK = LANES * NPHASE * NHIST // LANES


def _sc_body(seg_hbm, tgt_hbm, out_hbm, cnt_ref, sum_ref,
             seg0, seg1, tgt0, tgt1, red_cnt, red_sum, stage, sem0, sem1):
    img = lax.axis_index("s") * 2 + lax.axis_index("c")
    lane_base = lax.iota(jnp.int32, LANES) * (NPHASE * NHIST)
    ones = jnp.ones((LANES,), jnp.float32)
    zeros = jnp.zeros((LANES,), jnp.float32)
    segb = (seg0, seg1)
    tgtb = (tgt0, tgt1)
    sems = (sem0, sem1)

    def _start(w, b):
        c1 = pltpu.async_copy(seg_hbm.at[img, 0, pl.ds(w * WROW, WROW), :],
                              segb[b], sems[b])
        c2 = pltpu.async_copy(tgt_hbm.at[img, 0, pl.ds(w * WROW, WROW), :],
                              tgtb[b], sems[b])
        return c1, c2

    @plsc.parallel_loop(0, ZCHUNK, unroll=4)
    def _zero(i):
        cnt_ref[pl.ds(i * LANES, LANES)] = zeros
        sum_ref[pl.ds(i * LANES, LANES)] = zeros

    cur = _start(0, 0)
    nxt = _start(1, 1)
    macc = zeros
    for w in range(NWIN):
        b = w % 2
        cur[0].wait()
        cur[1].wait()
        seg_buf = segb[b]
        tgt_buf = tgtb[b]

        def _elem(i, macc):
            row = i >> 5
            col = (i & 31) * LANES
            lg = seg_buf[row, pl.ds(col, LANES)]
            tg = tgt_buf[row, pl.ds(col, LANES)]
            pos = tg > 0
            e = jnp.maximum(jnp.where(pos, 1.0 - lg, 1.0 + lg), 0.0)
            bk = (jnp.minimum(e, CLAMP) * SCALE).astype(jnp.int32)
            idx = lane_base + (i % NPHASE) * NHIST + bk + tg * NBE
            plsc.addupdate_scatter(cnt_ref, [idx], ones)
            plsc.addupdate_scatter(sum_ref, [idx], e)
            return jnp.maximum(macc, e)

        macc = plsc.parallel_loop(0, WIN // LANES, unroll=NPHASE,
                                  carry=macc)(_elem)
        cur = nxt
        if w + 2 < NWIN:
            nxt = _start(w + 2, b)

    def _reduce(i, gv):
        base = i * LANES
        cn = zeros
        sn = zeros
        for r in range(LANES * NPHASE):
            cn = cn + cnt_ref[pl.ds(r * NHIST + base, LANES)]
            sn = sn + sum_ref[pl.ds(r * NHIST + base, LANES)]
        red_cnt[pl.ds(base, LANES)] = cn
        red_sum[pl.ds(base, LANES)] = sn
        return gv + jnp.where(i >= NCHUNK, cn, zeros)

    gvec = lax.fori_loop(0, RCHUNK, _reduce, zeros)
    G = jnp.sum(gvec)

    def _chunk(i, carry):
        cm, cpc, lossacc = carry
        v0 = (NCHUNK - 1 - i) * LANES
        cn = red_cnt[pl.ds(v0, LANES)]
        cp = red_cnt[pl.ds(NBE + v0, LANES)]
        sn = red_sum[pl.ds(v0, LANES)]
        sp = red_sum[pl.ds(NBE + v0, LANES)]
        m = cn + cp
        pre_m = plsc.cumsum(m)
        pre_p = plsc.cumsum(cp)
        tot_m = jnp.sum(m)
        tot_p = jnp.sum(cp)
        nex = cm + (tot_m - pre_m)
        pex = cpc + (tot_p - pre_p)
        nb = nex - pex
        pos = sp / (G + nb + 0.5 * cn)
        neg = (G - pex - 0.5 * cp) * sn / ((G + nb) * (G + nb + cn))
        return cm + tot_m, cpc + tot_p, lossacc + pos + neg

    _, _, lossacc = lax.fori_loop(0, NCHUNK, _chunk, (0.0, 0.0, zeros))

    stage[pl.ds(0, LANES)] = lossacc
    stage[pl.ds(LANES, LANES)] = gvec
    stage[pl.ds(2 * LANES, LANES)] = macc
    pltpu.sync_copy(stage, out_hbm.at[img])


def _combine_body(sc_ref, pp_ref, o_ref):
    sc = sc_ref[...]
    lossv = jnp.sum(sc[:, 0:16], axis=1, keepdims=True)
    gv = jnp.sum(sc[:, 16:32], axis=1, keepdims=True)
    maxv = jnp.max(sc[:, 32:48], axis=1, keepdims=True)
    lov = jnp.where(gv > 0.0, lossv, maxv)
    lovasz = jnp.sum(lov) * (0.5 / B)
    p = jnp.clip(pp_ref[...], 1e-12, 1.0 - 1e-12)
    trg = (gv > 0.0).astype(jnp.float32)
    bce = -(trg * jnp.log(p) + (1.0 - trg) * jnp.log(1.0 - p))
    o_ref[...] = (jnp.sum(bce) * (0.5 / B) + lovasz).reshape(1, 1)


@jax.jit
def kernel(segm, prob_pred, target):
    seg2 = segm
    tgt2 = target.astype(jnp.int32)
    sc_fn = pl.kernel(
        _sc_body,
        out_type=jax.ShapeDtypeStruct((B, 3 * LANES), jnp.float32),
        mesh=plsc.VectorSubcoreMesh(core_axis_name="c", subcore_axis_name="s"),
        scratch_types=[
            pltpu.VMEM((LANES * NPHASE * NHIST,), jnp.float32),
            pltpu.VMEM((LANES * NPHASE * NHIST,), jnp.float32),
            pltpu.VMEM((WROW, 512), jnp.float32),
            pltpu.VMEM((WROW, 512), jnp.float32),
            pltpu.VMEM((WROW, 512), jnp.int32),
            pltpu.VMEM((WROW, 512), jnp.int32),
            pltpu.VMEM((NHIST,), jnp.float32),
            pltpu.VMEM((NHIST,), jnp.float32),
            pltpu.VMEM((3 * LANES,), jnp.float32),
            pltpu.SemaphoreType.DMA,
            pltpu.SemaphoreType.DMA,
        ],
        compiler_params=pltpu.CompilerParams(needs_layout_passes=False,
                                             use_tc_tiling_on_sc=True),
    )
    sc_out = sc_fn(seg2, tgt2)
    out = pl.pallas_call(
        _combine_body,
        out_shape=jax.ShapeDtypeStruct((1, 1), jnp.float32),
    )(sc_out, prob_pred.reshape(B, 1))
    return out[0, 0]
